# Initial kernel scaffold; baseline (speedup 1.0000x reference)
import jax
import jax.numpy as jnp
from jax.experimental import pallas as pl


def kernel(edge_index, src_emb, hr, weights_zero, weights_first, weights_middle, weights_last, W_zero, b_zero, g_zero, be_zero, W_first, b_first, g_first, be_first, W_middle, b_middle, g_middle, be_middle, W_last, b_last, g_last, be_last, W_cat, b_cat, g_h, be_h):
    raise NotImplementedError("write your pallas kernel here")


# speedup vs baseline: 2.5590x; 2.5590x over previous
import functools

import jax
import jax.numpy as jnp
from jax import lax
from jax.experimental import pallas as pl
from jax.experimental.pallas import tpu as pltpu
from jax.experimental.pallas import tpu_sc as plsc

N = 10000
E = 320000
D = 128

NT = 16
K = 128
C = 160
E_PAD = NT * C * K
N_PAD = 10112
RPT = N_PAD // 16

BLK = 1000
NB = N // BLK
NF = float(N)


def _sc_mesh():
    return plsc.VectorSubcoreMesh(core_axis_name="c", subcore_axis_name="s",
                                  num_cores=1)


def _segsum_dev(table, idx_p, zer):
    @functools.partial(
        pl.kernel,
        out_type=jax.ShapeDtypeStruct((N_PAD, D), jnp.float32),
        mesh=_sc_mesh(),
        scratch_types=[
            pltpu.VMEM((2, 2, K), jnp.int32),
            pltpu.VMEM((2, K, D), jnp.float32),
            pltpu.VMEM_SHARED((N_PAD, D), jnp.float32),
            pltpu.SemaphoreType.DMA((2,)),
            pltpu.SemaphoreType.DMA((2,)),
        ],
    )
    def seg(table_h, idx_h, zer_h, out_h, idx_v, rows_v, acc_s, isems, rsems):
        s = lax.axis_index("s")

        def issue_idx(ci, b):
            pltpu.async_copy(idx_h.at[s, ci], idx_v.at[b], isems.at[b])

        def wait_idx(ci, b):
            pltpu.make_async_copy(idx_h.at[s, ci], idx_v.at[b],
                                  isems.at[b]).wait()

        def issue_gather(ci, b):
            pltpu.async_copy(table_h.at[idx_v.at[b, 0]], rows_v.at[b],
                             rsems.at[b])

        def wait_gather(ci, b):
            pltpu.make_async_copy(table_h.at[idx_v.at[b, 0]], rows_v.at[b],
                                  rsems.at[b]).wait()

        issue_idx(0, 0)
        pltpu.sync_copy(zer_h, acc_s.at[pl.ds(s * RPT, RPT)])
        plsc.subcore_barrier()
        wait_idx(0, 0)
        issue_gather(0, 0)
        issue_idx(1, 1)

        def body(ci, carry):
            b = lax.rem(ci, 2)
            nb = lax.rem(ci + 1, 2)

            @pl.when(ci + 1 < C)
            def _():
                wait_idx(ci + 1, nb)
                issue_gather(ci + 1, nb)

            wait_gather(ci, b)
            pltpu.sync_copy(rows_v.at[b], acc_s.at[idx_v.at[b, 1]], add=True)

            @pl.when(ci + 2 < C)
            def _():
                issue_idx(ci + 2, b)

            return carry

        lax.fori_loop(0, C, body, 0)
        plsc.subcore_barrier()
        pltpu.sync_copy(acc_s.at[pl.ds(s * RPT, RPT)],
                        out_h.at[pl.ds(s * RPT, RPT)])

    return seg(table, idx_p, zer)


def _deg_dev(idx_p, ones_r, zer):
    @functools.partial(
        pl.kernel,
        out_type=jax.ShapeDtypeStruct((N_PAD, D), jnp.float32),
        mesh=_sc_mesh(),
        scratch_types=[
            pltpu.VMEM((2, 2, K), jnp.int32),
            pltpu.VMEM((K, D), jnp.float32),
            pltpu.VMEM_SHARED((N_PAD, D), jnp.float32),
            pltpu.SemaphoreType.DMA((2,)),
        ],
    )
    def deg(idx_h, ones_h, zer_h, out_h, idx_v, ones_v, acc_s, isems):
        s = lax.axis_index("s")

        def issue_idx(ci, b):
            pltpu.async_copy(idx_h.at[s, ci], idx_v.at[b], isems.at[b])

        def wait_idx(ci, b):
            pltpu.make_async_copy(idx_h.at[s, ci], idx_v.at[b],
                                  isems.at[b]).wait()

        issue_idx(0, 0)
        issue_idx(1, 1)
        pltpu.sync_copy(ones_h, ones_v)
        pltpu.sync_copy(zer_h, acc_s.at[pl.ds(s * RPT, RPT)])
        plsc.subcore_barrier()

        def body(ci, carry):
            b = lax.rem(ci, 2)
            wait_idx(ci, b)
            pltpu.sync_copy(ones_v, acc_s.at[idx_v.at[b, 1]], add=True)

            @pl.when(ci + 2 < C)
            def _():
                issue_idx(ci + 2, b)

            return carry

        lax.fori_loop(0, C, body, 0)
        plsc.subcore_barrier()
        pltpu.sync_copy(acc_s.at[pl.ds(s * RPT, RPT)],
                        out_h.at[pl.ds(s * RPT, RPT)])

    return deg(idx_p, ones_r, zer)


def _two_phase(p, i, ys, out_refs, acc_s, acc_q):
    @pl.when((p == 0) & (i == 0))
    def _():
        acc_s[...] = jnp.zeros_like(acc_s)
        acc_q[...] = jnp.zeros_like(acc_q)

    @pl.when(p == 0)
    def _():
        for j, (y, _, _, _, _) in enumerate(ys):
            acc_s[j:j + 1, :] += jnp.sum(y, axis=0, keepdims=True)
            acc_q[j:j + 1, :] += jnp.sum(y * y, axis=0, keepdims=True)

    @pl.when(p == 1)
    def _():
        outs = [jnp.zeros((BLK, D), jnp.float32) for _ in out_refs]
        for j, (y, g, be, w, slot) in enumerate(ys):
            m = acc_s[j:j + 1, :] / NF
            v = acc_q[j:j + 1, :] / NF - m * m
            yn = g * (y - m) * lax.rsqrt(v + 1e-5) + be
            outs[slot] = outs[slot] + w * jnp.maximum(yn, 0.0)
        for r, o in zip(out_refs, outs):
            r[...] = o


def _row_spec():
    return pl.BlockSpec((BLK, D), lambda p, i: (i, 0))


def _full_spec(shape):
    nd = len(shape)
    return pl.BlockSpec(shape, lambda p, i: (0,) * nd)


def _tc_params():
    return pltpu.CompilerParams(
        dimension_semantics=("arbitrary", "arbitrary"))


def _stage_zero(src_emb, hr, Wst, bst, gst, best, wz):
    n_y = 2

    def body(srcr, hrr, Wr, br, gr, ber, wr, outr, acc_s, acc_q):
        p = pl.program_id(0)
        i = pl.program_id(1)
        x0 = srcr[...] * hrr[...]
        x1 = srcr[...] + hrr[...]
        ys = []
        for k, x in enumerate((x0, x1)):
            y = jnp.dot(x, Wr[k], preferred_element_type=jnp.float32) + br[k]
            ys.append((y, gr[k], ber[k], wr[0, k], 0))
        _two_phase(p, i, ys, (outr,), acc_s, acc_q)

    return pl.pallas_call(
        body,
        grid=(2, NB),
        in_specs=[_row_spec(), _row_spec(), _full_spec(Wst.shape),
                  _full_spec(bst.shape), _full_spec(gst.shape),
                  _full_spec(best.shape),
                  pl.BlockSpec(memory_space=pltpu.SMEM)],
        out_specs=_row_spec(),
        out_shape=jax.ShapeDtypeStruct((N, D), jnp.float32),
        scratch_shapes=[pltpu.VMEM((8, D), jnp.float32),
                        pltpu.VMEM((8, D), jnp.float32)],
        compiler_params=_tc_params(),
    )(src_emb, hr, Wst, bst, gst, best, wz)


def _mixed_stage(states, parts, degs, Wst, bst, gst, best, wmat, nodes, n_out):
    ns = len(states)
    n_y = 3 * len(nodes)
    ny_pad = 8 * ((n_y + 7) // 8)

    def body(*refs):
        it = iter(refs)
        st = [next(it) for _ in range(ns)]
        P = [next(it) for _ in range(ns)]
        d0 = next(it)
        Wr = next(it)
        br = next(it)
        gr = next(it)
        ber = next(it)
        wr = next(it)
        outs = [next(it) for _ in range(n_out)]
        acc_s = next(it)
        acc_q = next(it)
        p = pl.program_id(0)
        i = pl.program_id(1)
        deg = jnp.maximum(d0[:, 0:1], 1.0)
        cache = {}

        def ops_for(si):
            if si not in cache:
                S = P[si][...]
                cache[si] = (S / deg, S, st[si][...])
            return cache[si]

        ys = []
        for (si, wbase, wrow, slot) in nodes:
            xs = ops_for(si)
            for k in range(3):
                y = jnp.dot(xs[k], Wr[wbase + k],
                            preferred_element_type=jnp.float32) + br[wbase + k]
                ys.append((y, gr[wbase + k], ber[wbase + k], wr[wrow, k], slot))
        _two_phase(p, i, ys, outs, acc_s, acc_q)

    in_specs = ([_row_spec()] * ns + [_row_spec()] * ns
                + [_row_spec()]
                + [_full_spec(Wst.shape), _full_spec(bst.shape),
                   _full_spec(gst.shape), _full_spec(best.shape),
                   pl.BlockSpec(memory_space=pltpu.SMEM)])
    out_shape = [jax.ShapeDtypeStruct((N, D), jnp.float32)] * n_out
    out_specs = [_row_spec()] * n_out
    args = (list(states) + list(parts)
            + [degs, Wst, bst, gst, best, wmat])
    res = pl.pallas_call(
        body,
        grid=(2, NB),
        in_specs=in_specs,
        out_specs=out_specs,
        out_shape=out_shape,
        scratch_shapes=[pltpu.VMEM((ny_pad, D), jnp.float32),
                        pltpu.VMEM((ny_pad, D), jnp.float32)],
        compiler_params=_tc_params(),
    )(*args)
    return res


def _stage_final(m1, m2, l1, l2, W_cat, b_cat, g_h, be_h):
    def body(r1, r2, r3, r4, Wr, br, gr, ber, outr, acc_s, acc_q):
        p = pl.program_id(0)
        i = pl.program_id(1)
        y = (jnp.dot(r1[...], Wr[0:D, :], preferred_element_type=jnp.float32)
             + jnp.dot(r2[...], Wr[D:2 * D, :], preferred_element_type=jnp.float32)
             + jnp.dot(r3[...], Wr[2 * D:3 * D, :], preferred_element_type=jnp.float32)
             + jnp.dot(r4[...], Wr[3 * D:4 * D, :], preferred_element_type=jnp.float32)
             + br[...])
        ys = [(y, gr[...], ber[...], 1.0, 0)]
        _two_phase(p, i, ys, (outr,), acc_s, acc_q)

    return pl.pallas_call(
        body,
        grid=(2, NB),
        in_specs=[_row_spec(), _row_spec(), _row_spec(), _row_spec(),
                  _full_spec(W_cat.shape), _full_spec(b_cat.shape),
                  _full_spec(g_h.shape), _full_spec(be_h.shape)],
        out_specs=_row_spec(),
        out_shape=jax.ShapeDtypeStruct((N, D), jnp.float32),
        scratch_shapes=[pltpu.VMEM((8, D), jnp.float32),
                        pltpu.VMEM((8, D), jnp.float32)],
        compiler_params=_tc_params(),
    )(m1, m2, l1, l2, W_cat, b_cat, g_h, be_h)


def kernel(edge_index, src_emb, hr, weights_zero, weights_first,
           weights_middle, weights_last, W_zero, b_zero, g_zero, be_zero,
           W_first, b_first, g_first, be_first, W_middle, b_middle, g_middle,
           be_middle, W_last, b_last, g_last, be_last, W_cat, b_cat, g_h,
           be_h):
    src = edge_index[0].astype(jnp.int32)
    dst = edge_index[1].astype(jnp.int32)
    src_p = jnp.concatenate(
        [src, jnp.zeros((E_PAD - E,), jnp.int32)]).reshape(NT, C, K)
    dst_p = jnp.concatenate(
        [dst, jnp.full((E_PAD - E,), N, jnp.int32)]).reshape(NT, C, K)
    idx_p = jnp.stack([src_p, dst_p], axis=2)
    zer = jnp.zeros((RPT, D), jnp.float32)

    degp = _deg_dev(idx_p, jnp.ones((K, D), jnp.float32), zer)

    h_in = _stage_zero(src_emb, hr, W_zero, b_zero, g_zero, be_zero,
                       weights_zero)

    Pin = _segsum_dev(h_in, idx_p, zer)

    s1 = _mixed_stage([h_in], [Pin], degp,
                      W_first[0:3], b_first[0:3], g_first[0:3], be_first[0:3],
                      weights_first[0:1], [(0, 0, 0, 0)], 1)[0]

    Ps1 = _segsum_dev(s1, idx_p, zer)

    Wc = jnp.concatenate([W_first[3:9], W_middle[0:3]], axis=0)
    bc = jnp.concatenate([b_first[3:9], b_middle[0:3]], axis=0)
    gc = jnp.concatenate([g_first[3:9], g_middle[0:3]], axis=0)
    bec = jnp.concatenate([be_first[3:9], be_middle[0:3]], axis=0)
    wc = jnp.concatenate([weights_first[1:3], weights_middle[0:1]], axis=0)
    s2, m1 = _mixed_stage([h_in, s1], [Pin, Ps1], degp, Wc, bc, gc, bec, wc,
                          [(0, 0, 0, 0), (1, 3, 1, 0), (1, 6, 2, 1)], 2)

    Ps2 = _segsum_dev(s2, idx_p, zer)

    m2 = _mixed_stage([s2], [Ps2], degp,
                      W_middle[3:6], b_middle[3:6], g_middle[3:6],
                      be_middle[3:6], weights_middle[1:2], [(0, 0, 0, 0)], 1)[0]

    Pm1 = _segsum_dev(m1, idx_p, zer)
    Pm2 = _segsum_dev(m2, idx_p, zer)

    l1 = _mixed_stage([m1, m2], [Pm1, Pm2], degp,
                      W_last[0:6], b_last[0:6], g_last[0:6], be_last[0:6],
                      weights_last[0:2], [(0, 0, 0, 0), (1, 3, 1, 0)], 1)[0]

    Pl1 = _segsum_dev(l1, idx_p, zer)

    l2 = _mixed_stage([m1, m2, l1], [Pm1, Pm2, Pl1], degp,
                      W_last[6:15], b_last[6:15], g_last[6:15], be_last[6:15],
                      weights_last[2:5],
                      [(0, 0, 0, 0), (1, 3, 1, 0), (2, 6, 2, 0)], 1)[0]

    return _stage_final(m1, m2, l1, l2, W_cat,
                        b_cat.reshape(1, D), g_h.reshape(1, D),
                        be_h.reshape(1, D))


# speedup vs baseline: 2.6396x; 1.0315x over previous
import functools

import jax
import jax.numpy as jnp
from jax import lax
from jax.experimental import pallas as pl
from jax.experimental.pallas import tpu as pltpu
from jax.experimental.pallas import tpu_sc as plsc

N = 10000
E = 320000
D = 128

NT = 16
K = 128
C = 160
E_PAD = NT * C * K
N_PAD = 10112
RPT = N_PAD // 16

BLK = 1000
NB = N // BLK
NF = float(N)


def _sc_mesh():
    return plsc.VectorSubcoreMesh(core_axis_name="c", subcore_axis_name="s",
                                  num_cores=1)


def _segsum_dev(table, idx_p, zer):
    @functools.partial(
        pl.kernel,
        out_type=jax.ShapeDtypeStruct((N_PAD, D), jnp.float32),
        mesh=_sc_mesh(),
        scratch_types=[
            pltpu.VMEM((2, 2, K), jnp.int32),
            pltpu.VMEM((3, K), jnp.int32),
            pltpu.VMEM((3, K, D), jnp.float32),
            pltpu.VMEM_SHARED((N_PAD, D), jnp.float32),
            pltpu.SemaphoreType.DMA((2,)),
            pltpu.SemaphoreType.DMA((3,)),
            pltpu.SemaphoreType.DMA((3,)),
        ],
    )
    def seg(table_h, idx_h, zer_h, out_h, idx_v, dst_v, rows_v, acc_s,
            isems, gsems, ssems):
        s = lax.axis_index("s")

        def issue_idx(ci, b):
            pltpu.async_copy(idx_h.at[s, ci], idx_v.at[b], isems.at[b])

        def wait_idx(ci, b):
            pltpu.make_async_copy(idx_h.at[s, ci], idx_v.at[b],
                                  isems.at[b]).wait()

        def issue_gather(b2, b3):
            pltpu.async_copy(table_h.at[idx_v.at[b2, 0]], rows_v.at[b3],
                             gsems.at[b3])

        def wait_gather(b2, b3):
            pltpu.make_async_copy(table_h.at[idx_v.at[b2, 0]], rows_v.at[b3],
                                  gsems.at[b3]).wait()

        def issue_scatter(b3):
            pltpu.async_copy(rows_v.at[b3], acc_s.at[dst_v.at[b3]],
                             ssems.at[b3], add=True)

        def wait_scatter(b3):
            pltpu.make_async_copy(rows_v.at[b3], acc_s.at[dst_v.at[b3]],
                                  ssems.at[b3]).wait()

        issue_idx(0, 0)
        issue_idx(1, 1)
        pltpu.sync_copy(zer_h, acc_s.at[pl.ds(s * RPT, RPT)])
        plsc.subcore_barrier()
        wait_idx(0, 0)
        issue_gather(0, 0)

        def body(ci, carry):
            b2 = lax.rem(ci, 2)
            b3 = lax.rem(ci, 3)

            @pl.when(ci >= 2)
            def _():
                wait_scatter(lax.rem(ci - 2, 3))

            @pl.when(ci + 1 < C)
            def _():
                wait_idx(ci + 1, lax.rem(ci + 1, 2))
                issue_gather(lax.rem(ci + 1, 2), lax.rem(ci + 1, 3))

            wait_gather(b2, b3)
            for j in range(K // 16):
                dst_v[b3, pl.ds(16 * j, 16)] = idx_v[b2, 1, pl.ds(16 * j, 16)]
            issue_scatter(b3)

            @pl.when(ci + 2 < C)
            def _():
                issue_idx(ci + 2, b2)

            return carry

        lax.fori_loop(0, C, body, 0)
        wait_scatter(lax.rem(C - 2, 3))
        wait_scatter(lax.rem(C - 1, 3))
        plsc.subcore_barrier()
        pltpu.sync_copy(acc_s.at[pl.ds(s * RPT, RPT)],
                        out_h.at[pl.ds(s * RPT, RPT)])

    return seg(table, idx_p, zer)


def _deg_dev(idx_p, ones_r, zer):
    @functools.partial(
        pl.kernel,
        out_type=jax.ShapeDtypeStruct((N_PAD, D), jnp.float32),
        mesh=_sc_mesh(),
        scratch_types=[
            pltpu.VMEM((2, 2, K), jnp.int32),
            pltpu.VMEM((K, D), jnp.float32),
            pltpu.VMEM_SHARED((N_PAD, D), jnp.float32),
            pltpu.SemaphoreType.DMA((2,)),
        ],
    )
    def deg(idx_h, ones_h, zer_h, out_h, idx_v, ones_v, acc_s, isems):
        s = lax.axis_index("s")

        def issue_idx(ci, b):
            pltpu.async_copy(idx_h.at[s, ci], idx_v.at[b], isems.at[b])

        def wait_idx(ci, b):
            pltpu.make_async_copy(idx_h.at[s, ci], idx_v.at[b],
                                  isems.at[b]).wait()

        issue_idx(0, 0)
        issue_idx(1, 1)
        pltpu.sync_copy(ones_h, ones_v)
        pltpu.sync_copy(zer_h, acc_s.at[pl.ds(s * RPT, RPT)])
        plsc.subcore_barrier()

        def body(ci, carry):
            b = lax.rem(ci, 2)
            wait_idx(ci, b)
            pltpu.sync_copy(ones_v, acc_s.at[idx_v.at[b, 1]], add=True)

            @pl.when(ci + 2 < C)
            def _():
                issue_idx(ci + 2, b)

            return carry

        lax.fori_loop(0, C, body, 0)
        plsc.subcore_barrier()
        pltpu.sync_copy(acc_s.at[pl.ds(s * RPT, RPT)],
                        out_h.at[pl.ds(s * RPT, RPT)])

    return deg(idx_p, ones_r, zer)


def _two_phase(p, i, ys, out_refs, acc_s, acc_q):
    @pl.when((p == 0) & (i == 0))
    def _():
        acc_s[...] = jnp.zeros_like(acc_s)
        acc_q[...] = jnp.zeros_like(acc_q)

    @pl.when(p == 0)
    def _():
        for j, (y, _, _, _, _) in enumerate(ys):
            acc_s[j:j + 1, :] += jnp.sum(y, axis=0, keepdims=True)
            acc_q[j:j + 1, :] += jnp.sum(y * y, axis=0, keepdims=True)

    @pl.when(p == 1)
    def _():
        outs = [jnp.zeros((BLK, D), jnp.float32) for _ in out_refs]
        for j, (y, g, be, w, slot) in enumerate(ys):
            m = acc_s[j:j + 1, :] / NF
            v = acc_q[j:j + 1, :] / NF - m * m
            yn = g * (y - m) * lax.rsqrt(v + 1e-5) + be
            outs[slot] = outs[slot] + w * jnp.maximum(yn, 0.0)
        for r, o in zip(out_refs, outs):
            r[...] = o


def _row_spec():
    return pl.BlockSpec((BLK, D), lambda p, i: (i, 0))


def _full_spec(shape):
    nd = len(shape)
    return pl.BlockSpec(shape, lambda p, i: (0,) * nd)


def _tc_params():
    return pltpu.CompilerParams(
        dimension_semantics=("arbitrary", "arbitrary"))


def _stage_zero(src_emb, hr, Wst, bst, gst, best, wz):
    n_y = 2

    def body(srcr, hrr, Wr, br, gr, ber, wr, outr, acc_s, acc_q):
        p = pl.program_id(0)
        i = pl.program_id(1)
        x0 = srcr[...] * hrr[...]
        x1 = srcr[...] + hrr[...]
        ys = []
        for k, x in enumerate((x0, x1)):
            y = jnp.dot(x, Wr[k], preferred_element_type=jnp.float32) + br[k]
            ys.append((y, gr[k], ber[k], wr[0, k], 0))
        _two_phase(p, i, ys, (outr,), acc_s, acc_q)

    return pl.pallas_call(
        body,
        grid=(2, NB),
        in_specs=[_row_spec(), _row_spec(), _full_spec(Wst.shape),
                  _full_spec(bst.shape), _full_spec(gst.shape),
                  _full_spec(best.shape),
                  pl.BlockSpec(memory_space=pltpu.SMEM)],
        out_specs=_row_spec(),
        out_shape=jax.ShapeDtypeStruct((N, D), jnp.float32),
        scratch_shapes=[pltpu.VMEM((8, D), jnp.float32),
                        pltpu.VMEM((8, D), jnp.float32)],
        compiler_params=_tc_params(),
    )(src_emb, hr, Wst, bst, gst, best, wz)


def _mixed_stage(states, parts, degs, Wst, bst, gst, best, wmat, nodes, n_out):
    ns = len(states)
    n_y = 3 * len(nodes)
    ny_pad = 8 * ((n_y + 7) // 8)

    def body(*refs):
        it = iter(refs)
        st = [next(it) for _ in range(ns)]
        P = [next(it) for _ in range(ns)]
        d0 = next(it)
        Wr = next(it)
        br = next(it)
        gr = next(it)
        ber = next(it)
        wr = next(it)
        outs = [next(it) for _ in range(n_out)]
        acc_s = next(it)
        acc_q = next(it)
        p = pl.program_id(0)
        i = pl.program_id(1)
        deg = jnp.maximum(d0[:, 0:1], 1.0)
        cache = {}

        def ops_for(si):
            if si not in cache:
                S = P[si][...]
                cache[si] = (S / deg, S, st[si][...])
            return cache[si]

        ys = []
        for (si, wbase, wrow, slot) in nodes:
            xs = ops_for(si)
            for k in range(3):
                y = jnp.dot(xs[k], Wr[wbase + k],
                            preferred_element_type=jnp.float32) + br[wbase + k]
                ys.append((y, gr[wbase + k], ber[wbase + k], wr[wrow, k], slot))
        _two_phase(p, i, ys, outs, acc_s, acc_q)

    in_specs = ([_row_spec()] * ns + [_row_spec()] * ns
                + [_row_spec()]
                + [_full_spec(Wst.shape), _full_spec(bst.shape),
                   _full_spec(gst.shape), _full_spec(best.shape),
                   pl.BlockSpec(memory_space=pltpu.SMEM)])
    out_shape = [jax.ShapeDtypeStruct((N, D), jnp.float32)] * n_out
    out_specs = [_row_spec()] * n_out
    args = (list(states) + list(parts)
            + [degs, Wst, bst, gst, best, wmat])
    res = pl.pallas_call(
        body,
        grid=(2, NB),
        in_specs=in_specs,
        out_specs=out_specs,
        out_shape=out_shape,
        scratch_shapes=[pltpu.VMEM((ny_pad, D), jnp.float32),
                        pltpu.VMEM((ny_pad, D), jnp.float32)],
        compiler_params=_tc_params(),
    )(*args)
    return res


def _stage_final(m1, m2, l1, l2, W_cat, b_cat, g_h, be_h):
    def body(r1, r2, r3, r4, Wr, br, gr, ber, outr, acc_s, acc_q):
        p = pl.program_id(0)
        i = pl.program_id(1)
        y = (jnp.dot(r1[...], Wr[0:D, :], preferred_element_type=jnp.float32)
             + jnp.dot(r2[...], Wr[D:2 * D, :], preferred_element_type=jnp.float32)
             + jnp.dot(r3[...], Wr[2 * D:3 * D, :], preferred_element_type=jnp.float32)
             + jnp.dot(r4[...], Wr[3 * D:4 * D, :], preferred_element_type=jnp.float32)
             + br[...])
        ys = [(y, gr[...], ber[...], 1.0, 0)]
        _two_phase(p, i, ys, (outr,), acc_s, acc_q)

    return pl.pallas_call(
        body,
        grid=(2, NB),
        in_specs=[_row_spec(), _row_spec(), _row_spec(), _row_spec(),
                  _full_spec(W_cat.shape), _full_spec(b_cat.shape),
                  _full_spec(g_h.shape), _full_spec(be_h.shape)],
        out_specs=_row_spec(),
        out_shape=jax.ShapeDtypeStruct((N, D), jnp.float32),
        scratch_shapes=[pltpu.VMEM((8, D), jnp.float32),
                        pltpu.VMEM((8, D), jnp.float32)],
        compiler_params=_tc_params(),
    )(m1, m2, l1, l2, W_cat, b_cat, g_h, be_h)


def kernel(edge_index, src_emb, hr, weights_zero, weights_first,
           weights_middle, weights_last, W_zero, b_zero, g_zero, be_zero,
           W_first, b_first, g_first, be_first, W_middle, b_middle, g_middle,
           be_middle, W_last, b_last, g_last, be_last, W_cat, b_cat, g_h,
           be_h):
    src = edge_index[0].astype(jnp.int32)
    dst = edge_index[1].astype(jnp.int32)
    src_p = jnp.concatenate(
        [src, jnp.zeros((E_PAD - E,), jnp.int32)]).reshape(NT, C, K)
    dst_p = jnp.concatenate(
        [dst, jnp.full((E_PAD - E,), N, jnp.int32)]).reshape(NT, C, K)
    idx_p = jnp.stack([src_p, dst_p], axis=2)
    zer = jnp.zeros((RPT, D), jnp.float32)

    degp = _deg_dev(idx_p, jnp.ones((K, D), jnp.float32), zer)

    h_in = _stage_zero(src_emb, hr, W_zero, b_zero, g_zero, be_zero,
                       weights_zero)

    Pin = _segsum_dev(h_in, idx_p, zer)

    s1 = _mixed_stage([h_in], [Pin], degp,
                      W_first[0:3], b_first[0:3], g_first[0:3], be_first[0:3],
                      weights_first[0:1], [(0, 0, 0, 0)], 1)[0]

    Ps1 = _segsum_dev(s1, idx_p, zer)

    Wc = jnp.concatenate([W_first[3:9], W_middle[0:3]], axis=0)
    bc = jnp.concatenate([b_first[3:9], b_middle[0:3]], axis=0)
    gc = jnp.concatenate([g_first[3:9], g_middle[0:3]], axis=0)
    bec = jnp.concatenate([be_first[3:9], be_middle[0:3]], axis=0)
    wc = jnp.concatenate([weights_first[1:3], weights_middle[0:1]], axis=0)
    s2, m1 = _mixed_stage([h_in, s1], [Pin, Ps1], degp, Wc, bc, gc, bec, wc,
                          [(0, 0, 0, 0), (1, 3, 1, 0), (1, 6, 2, 1)], 2)

    Ps2 = _segsum_dev(s2, idx_p, zer)

    m2 = _mixed_stage([s2], [Ps2], degp,
                      W_middle[3:6], b_middle[3:6], g_middle[3:6],
                      be_middle[3:6], weights_middle[1:2], [(0, 0, 0, 0)], 1)[0]

    Pm1 = _segsum_dev(m1, idx_p, zer)
    Pm2 = _segsum_dev(m2, idx_p, zer)

    l1 = _mixed_stage([m1, m2], [Pm1, Pm2], degp,
                      W_last[0:6], b_last[0:6], g_last[0:6], be_last[0:6],
                      weights_last[0:2], [(0, 0, 0, 0), (1, 3, 1, 0)], 1)[0]

    Pl1 = _segsum_dev(l1, idx_p, zer)

    l2 = _mixed_stage([m1, m2, l1], [Pm1, Pm2, Pl1], degp,
                      W_last[6:15], b_last[6:15], g_last[6:15], be_last[6:15],
                      weights_last[2:5],
                      [(0, 0, 0, 0), (1, 3, 1, 0), (2, 6, 2, 0)], 1)[0]

    return _stage_final(m1, m2, l1, l2, W_cat,
                        b_cat.reshape(1, D), g_h.reshape(1, D),
                        be_h.reshape(1, D))
